# baseline (speedup 1.0000x reference)
import jax
import jax.numpy as jnp
from jax.experimental import pallas as pl
from jax.experimental.pallas import tpu as pltpu


_TL = 512


def _add_body(x_ref, pe_ref, o_ref):
    o_ref[...] = x_ref[...] + pe_ref[...]


def kernel(x, pos_table):
    B, L, D = x.shape
    nblk = L // _TL
    return pl.pallas_call(
        _add_body,
        grid=(nblk, B),
        in_specs=[
            pl.BlockSpec((1, _TL, D), lambda l, b: (b, l, 0)),
            pl.BlockSpec((_TL, D), lambda l, b: (l, 0)),
        ],
        out_specs=pl.BlockSpec((1, _TL, D), lambda l, b: (b, l, 0)),
        out_shape=jax.ShapeDtypeStruct((B, L, D), x.dtype),
        compiler_params=pltpu.CompilerParams(
            dimension_semantics=("parallel", "parallel"),
        ),
    )(x, pos_table)


# speedup vs baseline: 1.1125x; 1.1125x over previous
import jax
import jax.numpy as jnp
from jax.experimental import pallas as pl
from jax.experimental.pallas import tpu as pltpu


_TL = 1024


def _add_body(x_ref, pe_ref, o_ref):
    o_ref[...] = x_ref[...] + pe_ref[...]


def kernel(x, pos_table):
    B, L, D = x.shape
    nblk = L // _TL
    return pl.pallas_call(
        _add_body,
        grid=(nblk, B),
        in_specs=[
            pl.BlockSpec((1, _TL, D), lambda l, b: (b, l, 0)),
            pl.BlockSpec((_TL, D), lambda l, b: (l, 0)),
        ],
        out_specs=pl.BlockSpec((1, _TL, D), lambda l, b: (b, l, 0)),
        out_shape=jax.ShapeDtypeStruct((B, L, D), x.dtype),
        compiler_params=pltpu.CompilerParams(
            dimension_semantics=("parallel", "parallel"),
        ),
    )(x, pos_table)


# speedup vs baseline: 1.1886x; 1.0684x over previous
import jax
import jax.numpy as jnp
from jax.experimental import pallas as pl
from jax.experimental.pallas import tpu as pltpu


_TL = 2048


def _add_body(x_ref, pe_ref, o_ref):
    o_ref[...] = x_ref[...] + pe_ref[...]


def kernel(x, pos_table):
    B, L, D = x.shape
    nblk = L // _TL
    return pl.pallas_call(
        _add_body,
        grid=(nblk, B),
        in_specs=[
            pl.BlockSpec((1, _TL, D), lambda l, b: (b, l, 0)),
            pl.BlockSpec((_TL, D), lambda l, b: (l, 0)),
        ],
        out_specs=pl.BlockSpec((1, _TL, D), lambda l, b: (b, l, 0)),
        out_shape=jax.ShapeDtypeStruct((B, L, D), x.dtype),
        compiler_params=pltpu.CompilerParams(
            dimension_semantics=("parallel", "parallel"),
        ),
    )(x, pos_table)
